# baseline (speedup 1.0000x reference)
import functools

import jax
import jax.numpy as jnp
from jax import lax
from jax.experimental import pallas as pl
from jax.experimental.pallas import tpu as pltpu
from jax.experimental.pallas import tpu_sc as plsc

N = 10000
E = 160000
IN_DIM = 256
HID = 512

NC = 2
NS = 16
CHUNK = 128
NPAD = 10240
EPAD = 163840
EDGES_PER_TILE = EPAD // NS
EDGES_PER_WORKER = EPAD // (NC * NS)
ROWS_PER_TILE = NPAD // NS
ZCOPIES = ROWS_PER_TILE // CHUNK
R = 640
F32 = jnp.float32
HIGHEST = lax.Precision.HIGHEST


def _mesh():
    return plsc.VectorSubcoreMesh(
        core_axis_name="c", subcore_axis_name="s",
        num_cores=NC, num_subcores=NS)


GTILE = EPAD // NS // CHUNK
GWORK = EPAD // (NC * NS) // CHUNK


def _zero_my_rows(z_v, acc, s):
    for j in range(ZCOPIES):
        pltpu.sync_copy(z_v, acc.at[pl.ds(s * ROWS_PER_TILE + j * CHUNK, CHUNK)])


def _dump_my_rows(acc, buf_v, out_hbm, s):
    for j in range(ZCOPIES):
        csl = pl.ds(s * ROWS_PER_TILE + j * CHUNK, CHUNK)
        pltpu.sync_copy(acc.at[csl], buf_v)
        pltpu.sync_copy(buf_v, out_hbm.at[csl])


def _spmv_scratch():
    return [
        pltpu.VMEM((GWORK, CHUNK), jnp.int32),
        pltpu.VMEM((GWORK, CHUNK), jnp.int32),
        pltpu.VMEM((CHUNK, CHUNK), F32),
        pltpu.VMEM((CHUNK, CHUNK), F32),
        pltpu.VMEM_SHARED((NPAD, CHUNK), F32),
        pltpu.SemaphoreType.DMA,
        pltpu.SemaphoreType.DMA,
        pltpu.SemaphoreType.DMA,
        pltpu.SemaphoreType.DMA,
    ]


def _edge_stream_pass(table_hbm, gat_hbm, sct_hbm, out_hbm, zeros_hbm, s,
                      first, nhalves, gidx2, sidx2, rows_a, rows_b, acc,
                      gs_a, gs_b, ss_a, ss_b):
    pltpu.sync_copy(zeros_hbm, rows_a)
    _zero_my_rows(rows_a, acc, s)
    plsc.subcore_barrier()

    def wait_gather(buf, sem):
        pltpu.make_async_copy(table_hbm.at[gidx2.at[0]], buf, sem).wait()

    def wait_scatter(buf, sem):
        pltpu.make_async_copy(buf, acc.at[sidx2.at[0]], sem).wait()

    nsteps = GWORK // 2

    def body(k, carry):
        g0 = 2 * k

        @pl.when(k > 0)
        def _():
            wait_scatter(rows_b, ss_b)

        pltpu.async_copy(table_hbm.at[gidx2.at[g0 + 1]], rows_b, gs_b)
        wait_gather(rows_a, gs_a)
        pltpu.async_copy(rows_a, acc.at[sidx2.at[g0]], ss_a, add=True)

        @pl.when(k < nsteps - 1)
        def _():
            wait_scatter(rows_a, ss_a)
            pltpu.async_copy(table_hbm.at[gidx2.at[g0 + 2]], rows_a, gs_a)

        wait_gather(rows_b, gs_b)
        pltpu.async_copy(rows_b, acc.at[sidx2.at[g0 + 1]], ss_b, add=True)
        return carry

    for half in range(nhalves):
        base = first + half * GWORK
        pltpu.sync_copy(gat_hbm.at[pl.ds(base, GWORK)], gidx2)
        pltpu.sync_copy(sct_hbm.at[pl.ds(base, GWORK)], sidx2)
        pltpu.async_copy(table_hbm.at[gidx2.at[0]], rows_a, gs_a)
        lax.fori_loop(0, nsteps, body, 0)
        wait_scatter(rows_a, ss_a)
        wait_scatter(rows_b, ss_b)

    plsc.subcore_barrier()
    _dump_my_rows(acc, rows_a, out_hbm, s)


@functools.cache
def _k1_degree():
    return pl.kernel(
        _k1_degree_body,
        out_type=[jax.ShapeDtypeStruct((NPAD, CHUNK), F32)] * 2,
        mesh=_mesh(),
        scratch_types=[
            pltpu.VMEM((GWORK, CHUNK), jnp.int32),
            pltpu.VMEM((CHUNK, CHUNK), F32),
            pltpu.VMEM((CHUNK, CHUNK), F32),
            pltpu.VMEM_SHARED((NPAD, CHUNK), F32),
            pltpu.SemaphoreType.DMA,
        ],
    )


def _k1_degree_body(dst2_hbm, ones_hbm, zeros_hbm, out0_hbm, out1_hbm,
                    didx2, ones_v, z_v, acc, ssem):
    c = lax.axis_index("c")
    s = lax.axis_index("s")
    first = (c * NS + s) * GWORK
    pltpu.sync_copy(dst2_hbm.at[pl.ds(first, GWORK)], didx2)
    pltpu.sync_copy(ones_hbm, ones_v)
    pltpu.sync_copy(zeros_hbm, z_v)
    _zero_my_rows(z_v, acc, s)
    plsc.subcore_barrier()

    def body(g, carry):
        pltpu.async_copy(ones_v, acc.at[didx2.at[g]], ssem, add=True)
        return carry

    lax.fori_loop(0, GWORK, body, 0)

    def drain(g, carry):
        pltpu.make_async_copy(ones_v, acc.at[didx2.at[0]], ssem).wait()
        return carry

    lax.fori_loop(0, GWORK, drain, 0)
    plsc.subcore_barrier()

    @pl.when(c == 0)
    def _():
        _dump_my_rows(acc, z_v, out0_hbm, s)

    @pl.when(c == 1)
    def _():
        _dump_my_rows(acc, z_v, out1_hbm, s)


def _spmv_pass(src2_hbm, dst2_hbm, table_hbm, out_hbm, zeros_hbm, s,
               gidx2, sidx2, rows_a, rows_b, acc, gs_a, gs_b, ss_a, ss_b):
    _edge_stream_pass(table_hbm, src2_hbm, dst2_hbm, out_hbm, zeros_hbm,
                      s, s * GTILE, GTILE // GWORK, gidx2, sidx2,
                      rows_a, rows_b, acc, gs_a, gs_b, ss_a, ss_b)


@functools.cache
def _k3_spmv2():
    return pl.kernel(
        _k3_spmv2_body,
        out_type=[jax.ShapeDtypeStruct((NPAD, CHUNK), F32)] * 4,
        mesh=_mesh(),
        scratch_types=_spmv_scratch(),
    )


def _k3_spmv2_body(src_hbm, dst_hbm, t0_hbm, t1_hbm, dinv_hbm, z_hbm,
                   o0_hbm, o1_hbm, cp0_hbm, cp1_hbm,
                   gidx2, sidx2, rows_a, rows_b, acc, gs_a, gs_b, ss_a, ss_b):
    c = lax.axis_index("c")
    s = lax.axis_index("s")
    firstw = (c * NS + s) * GWORK

    @pl.when(c == 0)
    def _():
        _spmv_pass(src_hbm, dst_hbm, t0_hbm, o0_hbm, z_hbm, s,
                   gidx2, sidx2, rows_a, rows_b, acc, gs_a, gs_b, ss_a, ss_b)
        _edge_stream_pass(dinv_hbm, dst_hbm, src_hbm, cp0_hbm, z_hbm,
                          s, firstw, 1, gidx2, sidx2, rows_a, rows_b, acc,
                          gs_a, gs_b, ss_a, ss_b)

    @pl.when(c == 1)
    def _():
        _spmv_pass(src_hbm, dst_hbm, t1_hbm, o1_hbm, z_hbm, s,
                   gidx2, sidx2, rows_a, rows_b, acc, gs_a, gs_b, ss_a, ss_b)
        _edge_stream_pass(dinv_hbm, dst_hbm, src_hbm, cp1_hbm, z_hbm,
                          s, firstw, 1, gidx2, sidx2, rows_a, rows_b, acc,
                          gs_a, gs_b, ss_a, ss_b)


@functools.cache
def _k5_spmv4():
    return pl.kernel(
        _k5_spmv4_body,
        out_type=[jax.ShapeDtypeStruct((NPAD, CHUNK), F32)] * 4,
        mesh=_mesh(),
        scratch_types=_spmv_scratch(),
    )


def _k5_spmv4_body(src_hbm, dst_hbm, t0_hbm, t1_hbm, t2_hbm, t3_hbm, z_hbm,
                   o0_hbm, o1_hbm, o2_hbm, o3_hbm,
                   gidx2, sidx2, rows_a, rows_b, acc, gs_a, gs_b, ss_a, ss_b):
    c = lax.axis_index("c")
    s = lax.axis_index("s")

    @pl.when(c == 0)
    def _():
        _spmv_pass(src_hbm, dst_hbm, t0_hbm, o0_hbm, z_hbm, s,
                   gidx2, sidx2, rows_a, rows_b, acc, gs_a, gs_b, ss_a, ss_b)
        _spmv_pass(src_hbm, dst_hbm, t1_hbm, o1_hbm, z_hbm, s,
                   gidx2, sidx2, rows_a, rows_b, acc, gs_a, gs_b, ss_a, ss_b)

    @pl.when(c == 1)
    def _():
        _spmv_pass(src_hbm, dst_hbm, t2_hbm, o2_hbm, z_hbm, s,
                   gidx2, sidx2, rows_a, rows_b, acc, gs_a, gs_b, ss_a, ss_b)
        _spmv_pass(src_hbm, dst_hbm, t3_hbm, o3_hbm, z_hbm, s,
                   gidx2, sidx2, rows_a, rows_b, acc, gs_a, gs_b, ss_a, ss_b)


def _k2_body(cnt0_ref, cnt1_ref, x_ref, dinv_ref, xs0_ref, xs1_ref):
    i = pl.program_id(0)
    cnt = cnt0_ref[:, 0:1] + cnt1_ref[:, 0:1]
    deg = cnt + 1.0
    y = lax.rsqrt(deg)
    dinv = y * (1.5 - 0.5 * deg * y * y)
    rows = i * R + lax.broadcasted_iota(jnp.int32, (R, 1), 0)
    dinv = jnp.where(rows < N, dinv, 0.0)
    dinv_ref[...] = jnp.broadcast_to(dinv, (R, CHUNK))
    xs = x_ref[...] * dinv
    xs0_ref[...] = xs[:, :CHUNK]
    xs1_ref[...] = xs[:, CHUNK:]


def _k2_call(cnt0, cnt1, xp):
    return pl.pallas_call(
        _k2_body,
        grid=(NPAD // R,),
        in_specs=[
            pl.BlockSpec((R, CHUNK), lambda i: (i, 0)),
            pl.BlockSpec((R, CHUNK), lambda i: (i, 0)),
            pl.BlockSpec((R, IN_DIM), lambda i: (i, 0)),
        ],
        out_specs=[
            pl.BlockSpec((R, CHUNK), lambda i: (i, 0)),
            pl.BlockSpec((R, CHUNK), lambda i: (i, 0)),
            pl.BlockSpec((R, CHUNK), lambda i: (i, 0)),
        ],
        out_shape=[
            jax.ShapeDtypeStruct((NPAD, CHUNK), F32),
            jax.ShapeDtypeStruct((NPAD, CHUNK), F32),
            jax.ShapeDtypeStruct((NPAD, CHUNK), F32),
        ],
    )(cnt0, cnt1, xp)


def _k4_body(p0_ref, p1_ref, xs0_ref, xs1_ref, dinv_ref, w1_ref, b1_ref,
             q0_ref, q1_ref, q2_ref, q3_ref):
    dinv = dinv_ref[:, 0:1]
    a = jnp.concatenate(
        [p0_ref[...] + xs0_ref[...], p1_ref[...] + xs1_ref[...]], axis=1)
    a = a * dinv
    h = lax.dot_general(a, w1_ref[...], (((1,), (0,)), ((), ())),
                        preferred_element_type=F32)
    h = jnp.maximum(h + b1_ref[...], 0.0)
    hs = h * dinv
    q0_ref[...] = hs[:, 0:128]
    q1_ref[...] = hs[:, 128:256]
    q2_ref[...] = hs[:, 256:384]
    q3_ref[...] = hs[:, 384:512]


def _k4_call(p0, p1, xs0, xs1, dinv, W1, b1r):
    return pl.pallas_call(
        _k4_body,
        grid=(NPAD // R,),
        in_specs=[
            pl.BlockSpec((R, CHUNK), lambda i: (i, 0)),
            pl.BlockSpec((R, CHUNK), lambda i: (i, 0)),
            pl.BlockSpec((R, CHUNK), lambda i: (i, 0)),
            pl.BlockSpec((R, CHUNK), lambda i: (i, 0)),
            pl.BlockSpec((R, CHUNK), lambda i: (i, 0)),
            pl.BlockSpec((IN_DIM, HID), lambda i: (0, 0)),
            pl.BlockSpec((1, HID), lambda i: (0, 0)),
        ],
        out_specs=[pl.BlockSpec((R, CHUNK), lambda i: (i, 0))] * 4,
        out_shape=[jax.ShapeDtypeStruct((NPAD, CHUNK), F32)] * 4,
    )(p0, p1, xs0, xs1, dinv, W1, b1r)


def _k6_body(m0_ref, m1_ref, m2_ref, m3_ref, q0_ref, q1_ref, q2_ref, q3_ref,
             dinv_ref, cp0_ref, cp1_ref, w2_ref, b2_ref, w3_ref, b3_ref,
             u_ref, racc):
    i = pl.program_id(0)
    dinv = dinv_ref[:, 0:1]
    a = jnp.concatenate([
        m0_ref[...] + q0_ref[...], m1_ref[...] + q1_ref[...],
        m2_ref[...] + q2_ref[...], m3_ref[...] + q3_ref[...]], axis=1)
    a = a * dinv
    h = lax.dot_general(a, w2_ref[...], (((1,), (0,)), ((), ())),
                        preferred_element_type=F32)
    h = jnp.maximum(h + b2_ref[...], 0.0)
    cvec = dinv * (cp0_ref[:, 0:1] + cp1_ref[:, 0:1] + dinv)
    part = lax.dot_general(cvec, h, (((0,), (0,)), ((), ())),
                           precision=HIGHEST, preferred_element_type=F32)

    @pl.when(i == 0)
    def _():
        racc[...] = part

    @pl.when(i > 0)
    def _():
        racc[...] = racc[...] + part

    @pl.when(i == pl.num_programs(0) - 1)
    def _():
        r = racc[...] * (1.0 / N)
        w3q = w3_ref[...].astype(jnp.bfloat16).astype(F32)
        u_ref[...] = lax.dot_general(r, w3q, (((1,), (0,)), ((), ())),
                                     precision=HIGHEST,
                                     preferred_element_type=F32) + b3_ref[...]


def _k6_call(m0, m1, m2, m3, q0, q1, q2, q3, dinv, cp0, cp1, W2, b2r, W3, b3r):
    return pl.pallas_call(
        _k6_body,
        grid=(NPAD // R,),
        in_specs=[pl.BlockSpec((R, CHUNK), lambda i: (i, 0))] * 8 + [
            pl.BlockSpec((R, CHUNK), lambda i: (i, 0)),
            pl.BlockSpec((R, CHUNK), lambda i: (i, 0)),
            pl.BlockSpec((R, CHUNK), lambda i: (i, 0)),
            pl.BlockSpec((HID, HID), lambda i: (0, 0)),
            pl.BlockSpec((1, HID), lambda i: (0, 0)),
            pl.BlockSpec((HID, HID), lambda i: (0, 0)),
            pl.BlockSpec((1, HID), lambda i: (0, 0)),
        ],
        out_specs=pl.BlockSpec((1, HID), lambda i: (0, 0)),
        out_shape=jax.ShapeDtypeStruct((1, HID), F32),
        scratch_shapes=[pltpu.VMEM((1, HID), F32)],
    )(m0, m1, m2, m3, q0, q1, q2, q3, dinv, cp0, cp1, W2, b2r, W3, b3r)


def kernel(x, edge_index, W1, b1, W2, b2, W3, b3):
    src = edge_index[0].astype(jnp.int32)
    dst = edge_index[1].astype(jnp.int32)
    padfill = jnp.full((EPAD - E,), N, jnp.int32)
    srcp = jnp.concatenate([src, padfill]).reshape(EPAD // CHUNK, CHUNK)
    dstp = jnp.concatenate([dst, padfill]).reshape(EPAD // CHUNK, CHUNK)
    xp = jnp.zeros((NPAD, IN_DIM), F32).at[:N].set(x)
    zeros128 = jnp.zeros((CHUNK, CHUNK), F32)
    ones128 = jnp.ones((CHUNK, CHUNK), F32)
    b1r = b1.reshape(1, HID)
    b2r = b2.reshape(1, HID)
    b3r = b3.reshape(1, HID)

    cnt0, cnt1 = _k1_degree()(dstp, ones128, zeros128)
    dinv, xs0, xs1 = _k2_call(cnt0, cnt1, xp)
    p0, p1, cp0, cp1 = _k3_spmv2()(srcp, dstp, xs0, xs1, dinv, zeros128)
    q0, q1, q2, q3 = _k4_call(p0, p1, xs0, xs1, dinv, W1, b1r)
    m0, m1, m2, m3 = _k5_spmv4()(srcp, dstp, q0, q1, q2, q3, zeros128)
    u = _k6_call(m0, m1, m2, m3, q0, q1, q2, q3, dinv, cp0, cp1,
                 W2, b2r, W3, b3r)
    return u


# speedup vs baseline: 1.0105x; 1.0105x over previous
import functools

import jax
import jax.numpy as jnp
from jax import lax
from jax.experimental import pallas as pl
from jax.experimental.pallas import tpu as pltpu
from jax.experimental.pallas import tpu_sc as plsc

N = 10000
E = 160000
IN_DIM = 256
HID = 512

NC = 2
NS = 16
CHUNK = 128
NPAD = 10240
EPAD = 163840
EDGES_PER_TILE = EPAD // NS
EDGES_PER_WORKER = EPAD // (NC * NS)
ROWS_PER_TILE = NPAD // NS
ZCOPIES = ROWS_PER_TILE // CHUNK
R = 640
F32 = jnp.float32
HIGHEST = lax.Precision.HIGHEST


def _mesh():
    return plsc.VectorSubcoreMesh(
        core_axis_name="c", subcore_axis_name="s",
        num_cores=NC, num_subcores=NS)


GTILE = EPAD // NS // CHUNK
GWORK = EPAD // (NC * NS) // CHUNK


def _zero_my_rows(z_v, acc, s):
    for j in range(ZCOPIES):
        pltpu.sync_copy(z_v, acc.at[pl.ds(s * ROWS_PER_TILE + j * CHUNK, CHUNK)])


def _dump_my_rows(acc, buf_v, out_hbm, s):
    for j in range(ZCOPIES):
        csl = pl.ds(s * ROWS_PER_TILE + j * CHUNK, CHUNK)
        pltpu.sync_copy(acc.at[csl], buf_v)
        pltpu.sync_copy(buf_v, out_hbm.at[csl])


def _spmv_scratch():
    return [
        pltpu.VMEM((GWORK, CHUNK), jnp.int32),
        pltpu.VMEM((GWORK, CHUNK), jnp.int32),
        pltpu.VMEM((CHUNK, CHUNK), F32),
        pltpu.VMEM((CHUNK, CHUNK), F32),
        pltpu.VMEM_SHARED((NPAD, CHUNK), F32),
        pltpu.SemaphoreType.DMA,
        pltpu.SemaphoreType.DMA,
        pltpu.SemaphoreType.DMA,
        pltpu.SemaphoreType.DMA,
    ]


def _edge_stream_pass(table_hbm, gat_hbm, sct_hbm, out_hbm, zeros_hbm, s,
                      first, nhalves, gidx2, sidx2, rows_a, rows_b, acc,
                      gs_a, gs_b, ss_a, ss_b):
    pltpu.sync_copy(zeros_hbm, rows_a)
    _zero_my_rows(rows_a, acc, s)
    plsc.subcore_barrier()

    def wait_gather(buf, sem):
        pltpu.make_async_copy(table_hbm.at[gidx2.at[0]], buf, sem).wait()

    def wait_scatter(buf, sem):
        pltpu.make_async_copy(buf, acc.at[sidx2.at[0]], sem).wait()

    nsteps = GWORK // 2

    def body(k, carry):
        g0 = 2 * k

        @pl.when(k > 0)
        def _():
            wait_scatter(rows_b, ss_b)

        pltpu.async_copy(table_hbm.at[gidx2.at[g0 + 1]], rows_b, gs_b)
        wait_gather(rows_a, gs_a)
        pltpu.async_copy(rows_a, acc.at[sidx2.at[g0]], ss_a, add=True)
        wait_gather(rows_b, gs_b)
        pltpu.async_copy(rows_b, acc.at[sidx2.at[g0 + 1]], ss_b, add=True)

        @pl.when(k < nsteps - 1)
        def _():
            wait_scatter(rows_a, ss_a)
            pltpu.async_copy(table_hbm.at[gidx2.at[g0 + 2]], rows_a, gs_a)

        return carry

    for half in range(nhalves):
        base = first + half * GWORK
        pltpu.sync_copy(gat_hbm.at[pl.ds(base, GWORK)], gidx2)
        pltpu.sync_copy(sct_hbm.at[pl.ds(base, GWORK)], sidx2)
        pltpu.async_copy(table_hbm.at[gidx2.at[0]], rows_a, gs_a)
        lax.fori_loop(0, nsteps, body, 0)
        wait_scatter(rows_a, ss_a)
        wait_scatter(rows_b, ss_b)

    plsc.subcore_barrier()
    _dump_my_rows(acc, rows_a, out_hbm, s)


@functools.cache
def _k1_degree():
    return pl.kernel(
        _k1_degree_body,
        out_type=[jax.ShapeDtypeStruct((NPAD, CHUNK), F32)] * 2,
        mesh=_mesh(),
        scratch_types=[
            pltpu.VMEM((GWORK, CHUNK), jnp.int32),
            pltpu.VMEM((CHUNK, CHUNK), F32),
            pltpu.VMEM((CHUNK, CHUNK), F32),
            pltpu.VMEM_SHARED((NPAD, CHUNK), F32),
            pltpu.SemaphoreType.DMA,
        ],
    )


def _k1_degree_body(dst2_hbm, ones_hbm, zeros_hbm, out0_hbm, out1_hbm,
                    didx2, ones_v, z_v, acc, ssem):
    c = lax.axis_index("c")
    s = lax.axis_index("s")
    first = (c * NS + s) * GWORK
    pltpu.sync_copy(dst2_hbm.at[pl.ds(first, GWORK)], didx2)
    pltpu.sync_copy(ones_hbm, ones_v)
    pltpu.sync_copy(zeros_hbm, z_v)
    _zero_my_rows(z_v, acc, s)
    plsc.subcore_barrier()

    def body(g, carry):
        pltpu.async_copy(ones_v, acc.at[didx2.at[g]], ssem, add=True)
        return carry

    lax.fori_loop(0, GWORK, body, 0)

    def drain(g, carry):
        pltpu.make_async_copy(ones_v, acc.at[didx2.at[0]], ssem).wait()
        return carry

    lax.fori_loop(0, GWORK, drain, 0)
    plsc.subcore_barrier()

    @pl.when(c == 0)
    def _():
        _dump_my_rows(acc, z_v, out0_hbm, s)

    @pl.when(c == 1)
    def _():
        _dump_my_rows(acc, z_v, out1_hbm, s)


@functools.cache
def _kc_cvec():
    return pl.kernel(
        _kc_cvec_body,
        out_type=[jax.ShapeDtypeStruct((NPAD, CHUNK), F32)] * 2,
        mesh=_mesh(),
        scratch_types=_spmv_scratch(),
    )


def _kc_cvec_body(src2_hbm, dst2_hbm, dinv_hbm, zeros_hbm, out0_hbm, out1_hbm,
                  gidx2, sidx2, rows_a, rows_b, acc, gs_a, gs_b, ss_a, ss_b):
    c = lax.axis_index("c")
    s = lax.axis_index("s")
    first = (c * NS + s) * GWORK

    @pl.when(c == 0)
    def _():
        _edge_stream_pass(dinv_hbm, dst2_hbm, src2_hbm, out0_hbm, zeros_hbm,
                          s, first, 1, gidx2, sidx2, rows_a, rows_b, acc,
                          gs_a, gs_b, ss_a, ss_b)

    @pl.when(c == 1)
    def _():
        _edge_stream_pass(dinv_hbm, dst2_hbm, src2_hbm, out1_hbm, zeros_hbm,
                          s, first, 1, gidx2, sidx2, rows_a, rows_b, acc,
                          gs_a, gs_b, ss_a, ss_b)


def _spmv_pass(src2_hbm, dst2_hbm, table_hbm, out_hbm, zeros_hbm, s,
               gidx2, sidx2, rows_a, rows_b, acc, gs_a, gs_b, ss_a, ss_b):
    _edge_stream_pass(table_hbm, src2_hbm, dst2_hbm, out_hbm, zeros_hbm,
                      s, s * GTILE, GTILE // GWORK, gidx2, sidx2,
                      rows_a, rows_b, acc, gs_a, gs_b, ss_a, ss_b)


@functools.cache
def _k3_spmv2():
    return pl.kernel(
        _k3_spmv2_body,
        out_type=[jax.ShapeDtypeStruct((NPAD, CHUNK), F32)] * 2,
        mesh=_mesh(),
        scratch_types=_spmv_scratch(),
    )


def _k3_spmv2_body(src_hbm, dst_hbm, t0_hbm, t1_hbm, z_hbm, o0_hbm, o1_hbm,
                   gidx2, sidx2, rows_a, rows_b, acc, gs_a, gs_b, ss_a, ss_b):
    c = lax.axis_index("c")
    s = lax.axis_index("s")

    @pl.when(c == 0)
    def _():
        _spmv_pass(src_hbm, dst_hbm, t0_hbm, o0_hbm, z_hbm, s,
                   gidx2, sidx2, rows_a, rows_b, acc, gs_a, gs_b, ss_a, ss_b)

    @pl.when(c == 1)
    def _():
        _spmv_pass(src_hbm, dst_hbm, t1_hbm, o1_hbm, z_hbm, s,
                   gidx2, sidx2, rows_a, rows_b, acc, gs_a, gs_b, ss_a, ss_b)


@functools.cache
def _k5_spmv4():
    return pl.kernel(
        _k5_spmv4_body,
        out_type=[jax.ShapeDtypeStruct((NPAD, CHUNK), F32)] * 4,
        mesh=_mesh(),
        scratch_types=_spmv_scratch(),
    )


def _k5_spmv4_body(src_hbm, dst_hbm, t0_hbm, t1_hbm, t2_hbm, t3_hbm, z_hbm,
                   o0_hbm, o1_hbm, o2_hbm, o3_hbm,
                   gidx2, sidx2, rows_a, rows_b, acc, gs_a, gs_b, ss_a, ss_b):
    c = lax.axis_index("c")
    s = lax.axis_index("s")

    @pl.when(c == 0)
    def _():
        _spmv_pass(src_hbm, dst_hbm, t0_hbm, o0_hbm, z_hbm, s,
                   gidx2, sidx2, rows_a, rows_b, acc, gs_a, gs_b, ss_a, ss_b)
        _spmv_pass(src_hbm, dst_hbm, t1_hbm, o1_hbm, z_hbm, s,
                   gidx2, sidx2, rows_a, rows_b, acc, gs_a, gs_b, ss_a, ss_b)

    @pl.when(c == 1)
    def _():
        _spmv_pass(src_hbm, dst_hbm, t2_hbm, o2_hbm, z_hbm, s,
                   gidx2, sidx2, rows_a, rows_b, acc, gs_a, gs_b, ss_a, ss_b)
        _spmv_pass(src_hbm, dst_hbm, t3_hbm, o3_hbm, z_hbm, s,
                   gidx2, sidx2, rows_a, rows_b, acc, gs_a, gs_b, ss_a, ss_b)


def _k2_body(cnt0_ref, cnt1_ref, x_ref, dinv_ref, xs0_ref, xs1_ref):
    i = pl.program_id(0)
    cnt = cnt0_ref[:, 0:1] + cnt1_ref[:, 0:1]
    deg = cnt + 1.0
    y = lax.rsqrt(deg)
    dinv = y * (1.5 - 0.5 * deg * y * y)
    rows = i * R + lax.broadcasted_iota(jnp.int32, (R, 1), 0)
    dinv = jnp.where(rows < N, dinv, 0.0)
    dinv_ref[...] = jnp.broadcast_to(dinv, (R, CHUNK))
    xs = x_ref[...] * dinv
    xs0_ref[...] = xs[:, :CHUNK]
    xs1_ref[...] = xs[:, CHUNK:]


def _k2_call(cnt0, cnt1, xp):
    return pl.pallas_call(
        _k2_body,
        grid=(NPAD // R,),
        in_specs=[
            pl.BlockSpec((R, CHUNK), lambda i: (i, 0)),
            pl.BlockSpec((R, CHUNK), lambda i: (i, 0)),
            pl.BlockSpec((R, IN_DIM), lambda i: (i, 0)),
        ],
        out_specs=[
            pl.BlockSpec((R, CHUNK), lambda i: (i, 0)),
            pl.BlockSpec((R, CHUNK), lambda i: (i, 0)),
            pl.BlockSpec((R, CHUNK), lambda i: (i, 0)),
        ],
        out_shape=[
            jax.ShapeDtypeStruct((NPAD, CHUNK), F32),
            jax.ShapeDtypeStruct((NPAD, CHUNK), F32),
            jax.ShapeDtypeStruct((NPAD, CHUNK), F32),
        ],
    )(cnt0, cnt1, xp)


def _k4_body(p0_ref, p1_ref, xs0_ref, xs1_ref, dinv_ref, w1_ref, b1_ref,
             q0_ref, q1_ref, q2_ref, q3_ref):
    dinv = dinv_ref[:, 0:1]
    a = jnp.concatenate(
        [p0_ref[...] + xs0_ref[...], p1_ref[...] + xs1_ref[...]], axis=1)
    a = a * dinv
    h = lax.dot_general(a, w1_ref[...], (((1,), (0,)), ((), ())),
                        preferred_element_type=F32)
    h = jnp.maximum(h + b1_ref[...], 0.0)
    hs = h * dinv
    q0_ref[...] = hs[:, 0:128]
    q1_ref[...] = hs[:, 128:256]
    q2_ref[...] = hs[:, 256:384]
    q3_ref[...] = hs[:, 384:512]


def _k4_call(p0, p1, xs0, xs1, dinv, W1, b1r):
    return pl.pallas_call(
        _k4_body,
        grid=(NPAD // R,),
        in_specs=[
            pl.BlockSpec((R, CHUNK), lambda i: (i, 0)),
            pl.BlockSpec((R, CHUNK), lambda i: (i, 0)),
            pl.BlockSpec((R, CHUNK), lambda i: (i, 0)),
            pl.BlockSpec((R, CHUNK), lambda i: (i, 0)),
            pl.BlockSpec((R, CHUNK), lambda i: (i, 0)),
            pl.BlockSpec((IN_DIM, HID), lambda i: (0, 0)),
            pl.BlockSpec((1, HID), lambda i: (0, 0)),
        ],
        out_specs=[pl.BlockSpec((R, CHUNK), lambda i: (i, 0))] * 4,
        out_shape=[jax.ShapeDtypeStruct((NPAD, CHUNK), F32)] * 4,
    )(p0, p1, xs0, xs1, dinv, W1, b1r)


def _k6_body(m0_ref, m1_ref, m2_ref, m3_ref, q0_ref, q1_ref, q2_ref, q3_ref,
             dinv_ref, cp0_ref, cp1_ref, w2_ref, b2_ref, w3_ref, b3_ref,
             u_ref, racc):
    i = pl.program_id(0)
    dinv = dinv_ref[:, 0:1]
    a = jnp.concatenate([
        m0_ref[...] + q0_ref[...], m1_ref[...] + q1_ref[...],
        m2_ref[...] + q2_ref[...], m3_ref[...] + q3_ref[...]], axis=1)
    a = a * dinv
    h = lax.dot_general(a, w2_ref[...], (((1,), (0,)), ((), ())),
                        preferred_element_type=F32)
    h = jnp.maximum(h + b2_ref[...], 0.0)
    cvec = dinv * (cp0_ref[:, 0:1] + cp1_ref[:, 0:1] + dinv)
    part = lax.dot_general(cvec, h, (((0,), (0,)), ((), ())),
                           precision=HIGHEST, preferred_element_type=F32)

    @pl.when(i == 0)
    def _():
        racc[...] = part

    @pl.when(i > 0)
    def _():
        racc[...] = racc[...] + part

    @pl.when(i == pl.num_programs(0) - 1)
    def _():
        r = racc[...] * (1.0 / N)
        w3q = w3_ref[...].astype(jnp.bfloat16).astype(F32)
        u_ref[...] = lax.dot_general(r, w3q, (((1,), (0,)), ((), ())),
                                     precision=HIGHEST,
                                     preferred_element_type=F32) + b3_ref[...]


def _k6_call(m0, m1, m2, m3, q0, q1, q2, q3, dinv, cp0, cp1, W2, b2r, W3, b3r):
    return pl.pallas_call(
        _k6_body,
        grid=(NPAD // R,),
        in_specs=[pl.BlockSpec((R, CHUNK), lambda i: (i, 0))] * 8 + [
            pl.BlockSpec((R, CHUNK), lambda i: (i, 0)),
            pl.BlockSpec((R, CHUNK), lambda i: (i, 0)),
            pl.BlockSpec((R, CHUNK), lambda i: (i, 0)),
            pl.BlockSpec((HID, HID), lambda i: (0, 0)),
            pl.BlockSpec((1, HID), lambda i: (0, 0)),
            pl.BlockSpec((HID, HID), lambda i: (0, 0)),
            pl.BlockSpec((1, HID), lambda i: (0, 0)),
        ],
        out_specs=pl.BlockSpec((1, HID), lambda i: (0, 0)),
        out_shape=jax.ShapeDtypeStruct((1, HID), F32),
        scratch_shapes=[pltpu.VMEM((1, HID), F32)],
    )(m0, m1, m2, m3, q0, q1, q2, q3, dinv, cp0, cp1, W2, b2r, W3, b3r)


def kernel(x, edge_index, W1, b1, W2, b2, W3, b3):
    src = edge_index[0].astype(jnp.int32)
    dst = edge_index[1].astype(jnp.int32)
    padfill = jnp.full((EPAD - E,), N, jnp.int32)
    srcp = jnp.concatenate([src, padfill]).reshape(EPAD // CHUNK, CHUNK)
    dstp = jnp.concatenate([dst, padfill]).reshape(EPAD // CHUNK, CHUNK)
    xp = jnp.zeros((NPAD, IN_DIM), F32).at[:N].set(x)
    zeros128 = jnp.zeros((CHUNK, CHUNK), F32)
    ones128 = jnp.ones((CHUNK, CHUNK), F32)
    b1r = b1.reshape(1, HID)
    b2r = b2.reshape(1, HID)
    b3r = b3.reshape(1, HID)

    cnt0, cnt1 = _k1_degree()(dstp, ones128, zeros128)
    dinv, xs0, xs1 = _k2_call(cnt0, cnt1, xp)
    p0, p1 = _k3_spmv2()(srcp, dstp, xs0, xs1, zeros128)
    cp0, cp1 = _kc_cvec()(srcp, dstp, dinv, zeros128)
    q0, q1, q2, q3 = _k4_call(p0, p1, xs0, xs1, dinv, W1, b1r)
    m0, m1, m2, m3 = _k5_spmv4()(srcp, dstp, q0, q1, q2, q3, zeros128)
    u = _k6_call(m0, m1, m2, m3, q0, q1, q2, q3, dinv, cp0, cp1,
                 W2, b2r, W3, b3r)
    return u


# speedup vs baseline: 1.0791x; 1.0678x over previous
import functools

import jax
import jax.numpy as jnp
from jax import lax
from jax.experimental import pallas as pl
from jax.experimental.pallas import tpu as pltpu
from jax.experimental.pallas import tpu_sc as plsc

N = 10000
E = 160000
IN_DIM = 256
HID = 512

NC = 2
NS = 16
CHUNK = 128
NPAD = 10240
EPAD = 163840
EDGES_PER_TILE = EPAD // NS
EDGES_PER_WORKER = EPAD // (NC * NS)
ROWS_PER_TILE = NPAD // NS
ZCOPIES = ROWS_PER_TILE // CHUNK
R = 640
F32 = jnp.float32
HIGHEST = lax.Precision.HIGHEST


def _mesh():
    return plsc.VectorSubcoreMesh(
        core_axis_name="c", subcore_axis_name="s",
        num_cores=NC, num_subcores=NS)


GTILE = EPAD // NS // CHUNK
GWORK = EPAD // (NC * NS) // CHUNK


def _zero_my_rows(z_v, acc, s):
    for j in range(ZCOPIES):
        pltpu.sync_copy(z_v, acc.at[pl.ds(s * ROWS_PER_TILE + j * CHUNK, CHUNK)])


def _dump_my_rows(acc, buf_v, out_hbm, s):
    for j in range(ZCOPIES):
        csl = pl.ds(s * ROWS_PER_TILE + j * CHUNK, CHUNK)
        pltpu.sync_copy(acc.at[csl], buf_v)
        pltpu.sync_copy(buf_v, out_hbm.at[csl])


def _spmv_scratch():
    return [
        pltpu.VMEM((GWORK, CHUNK), jnp.int32),
        pltpu.VMEM((GWORK, CHUNK), jnp.int32),
        pltpu.VMEM((CHUNK, CHUNK), F32),
        pltpu.VMEM((CHUNK, CHUNK), F32),
        pltpu.VMEM_SHARED((NPAD, CHUNK), F32),
        pltpu.SemaphoreType.DMA,
        pltpu.SemaphoreType.DMA,
        pltpu.SemaphoreType.DMA,
        pltpu.SemaphoreType.DMA,
    ]


def _edge_stream_pass(table_hbm, gat_hbm, sct_hbm, out_hbm, zeros_hbm, s,
                      first, nhalves, gidx2, sidx2, rows_a, rows_b, acc,
                      gs_a, gs_b, ss_a, ss_b):
    pltpu.sync_copy(zeros_hbm, rows_a)
    _zero_my_rows(rows_a, acc, s)
    plsc.subcore_barrier()

    def wait_gather(buf, sem):
        pltpu.make_async_copy(table_hbm.at[gidx2.at[0]], buf, sem).wait()

    def wait_scatter(buf, sem):
        pltpu.make_async_copy(buf, acc.at[sidx2.at[0]], sem).wait()

    nsteps = GWORK // 2

    def body(k, carry):
        g0 = 2 * k

        @pl.when(k > 0)
        def _():
            wait_scatter(rows_b, ss_b)

        pltpu.async_copy(table_hbm.at[gidx2.at[g0 + 1]], rows_b, gs_b)
        wait_gather(rows_a, gs_a)
        pltpu.async_copy(rows_a, acc.at[sidx2.at[g0]], ss_a, add=True)

        @pl.when(k < nsteps - 1)
        def _():
            wait_scatter(rows_a, ss_a)
            pltpu.async_copy(table_hbm.at[gidx2.at[g0 + 2]], rows_a, gs_a)

        wait_gather(rows_b, gs_b)
        pltpu.async_copy(rows_b, acc.at[sidx2.at[g0 + 1]], ss_b, add=True)
        return carry

    for half in range(nhalves):
        base = first + half * GWORK
        pltpu.sync_copy(gat_hbm.at[pl.ds(base, GWORK)], gidx2)
        pltpu.sync_copy(sct_hbm.at[pl.ds(base, GWORK)], sidx2)
        pltpu.async_copy(table_hbm.at[gidx2.at[0]], rows_a, gs_a)
        lax.fori_loop(0, nsteps, body, 0)
        wait_scatter(rows_a, ss_a)
        wait_scatter(rows_b, ss_b)

    plsc.subcore_barrier()
    _dump_my_rows(acc, rows_a, out_hbm, s)


@functools.cache
def _k1_degree():
    return pl.kernel(
        _k1_degree_body,
        out_type=[jax.ShapeDtypeStruct((NPAD, CHUNK), F32)] * 2,
        mesh=_mesh(),
        scratch_types=[
            pltpu.VMEM((GWORK, CHUNK), jnp.int32),
            pltpu.VMEM((CHUNK, CHUNK), F32),
            pltpu.VMEM((CHUNK, CHUNK), F32),
            pltpu.VMEM_SHARED((NPAD, CHUNK), F32),
            pltpu.SemaphoreType.DMA,
        ],
    )


def _k1_degree_body(dst2_hbm, ones_hbm, zeros_hbm, out0_hbm, out1_hbm,
                    didx2, ones_v, z_v, acc, ssem):
    c = lax.axis_index("c")
    s = lax.axis_index("s")
    first = (c * NS + s) * GWORK
    pltpu.sync_copy(dst2_hbm.at[pl.ds(first, GWORK)], didx2)
    pltpu.sync_copy(ones_hbm, ones_v)
    pltpu.sync_copy(zeros_hbm, z_v)
    _zero_my_rows(z_v, acc, s)
    plsc.subcore_barrier()

    def body(g, carry):
        pltpu.async_copy(ones_v, acc.at[didx2.at[g]], ssem, add=True)
        return carry

    lax.fori_loop(0, GWORK, body, 0)

    def drain(g, carry):
        pltpu.make_async_copy(ones_v, acc.at[didx2.at[0]], ssem).wait()
        return carry

    lax.fori_loop(0, GWORK, drain, 0)
    plsc.subcore_barrier()

    @pl.when(c == 0)
    def _():
        _dump_my_rows(acc, z_v, out0_hbm, s)

    @pl.when(c == 1)
    def _():
        _dump_my_rows(acc, z_v, out1_hbm, s)


@functools.cache
def _kc_cvec():
    return pl.kernel(
        _kc_cvec_body,
        out_type=[jax.ShapeDtypeStruct((NPAD, CHUNK), F32)] * 2,
        mesh=_mesh(),
        scratch_types=_spmv_scratch(),
    )


def _kc_cvec_body(src2_hbm, dst2_hbm, dinv_hbm, zeros_hbm, out0_hbm, out1_hbm,
                  gidx2, sidx2, rows_a, rows_b, acc, gs_a, gs_b, ss_a, ss_b):
    c = lax.axis_index("c")
    s = lax.axis_index("s")
    first = (c * NS + s) * GWORK

    @pl.when(c == 0)
    def _():
        _edge_stream_pass(dinv_hbm, dst2_hbm, src2_hbm, out0_hbm, zeros_hbm,
                          s, first, 1, gidx2, sidx2, rows_a, rows_b, acc,
                          gs_a, gs_b, ss_a, ss_b)

    @pl.when(c == 1)
    def _():
        _edge_stream_pass(dinv_hbm, dst2_hbm, src2_hbm, out1_hbm, zeros_hbm,
                          s, first, 1, gidx2, sidx2, rows_a, rows_b, acc,
                          gs_a, gs_b, ss_a, ss_b)


def _spmv_pass(src2_hbm, dst2_hbm, table_hbm, out_hbm, zeros_hbm, s,
               gidx2, sidx2, rows_a, rows_b, acc, gs_a, gs_b, ss_a, ss_b):
    _edge_stream_pass(table_hbm, src2_hbm, dst2_hbm, out_hbm, zeros_hbm,
                      s, s * GTILE, GTILE // GWORK, gidx2, sidx2,
                      rows_a, rows_b, acc, gs_a, gs_b, ss_a, ss_b)


@functools.cache
def _k3_spmv2():
    return pl.kernel(
        _k3_spmv2_body,
        out_type=[jax.ShapeDtypeStruct((NPAD, CHUNK), F32)] * 2,
        mesh=_mesh(),
        scratch_types=_spmv_scratch(),
    )


def _k3_spmv2_body(src_hbm, dst_hbm, t0_hbm, t1_hbm, z_hbm, o0_hbm, o1_hbm,
                   gidx2, sidx2, rows_a, rows_b, acc, gs_a, gs_b, ss_a, ss_b):
    c = lax.axis_index("c")
    s = lax.axis_index("s")

    @pl.when(c == 0)
    def _():
        _spmv_pass(src_hbm, dst_hbm, t0_hbm, o0_hbm, z_hbm, s,
                   gidx2, sidx2, rows_a, rows_b, acc, gs_a, gs_b, ss_a, ss_b)

    @pl.when(c == 1)
    def _():
        _spmv_pass(src_hbm, dst_hbm, t1_hbm, o1_hbm, z_hbm, s,
                   gidx2, sidx2, rows_a, rows_b, acc, gs_a, gs_b, ss_a, ss_b)


@functools.cache
def _k5_spmv4():
    return pl.kernel(
        _k5_spmv4_body,
        out_type=[jax.ShapeDtypeStruct((NPAD, CHUNK), F32)] * 4,
        mesh=_mesh(),
        scratch_types=_spmv_scratch(),
    )


def _k5_spmv4_body(src_hbm, dst_hbm, t0_hbm, t1_hbm, t2_hbm, t3_hbm, z_hbm,
                   o0_hbm, o1_hbm, o2_hbm, o3_hbm,
                   gidx2, sidx2, rows_a, rows_b, acc, gs_a, gs_b, ss_a, ss_b):
    c = lax.axis_index("c")
    s = lax.axis_index("s")

    @pl.when(c == 0)
    def _():
        _spmv_pass(src_hbm, dst_hbm, t0_hbm, o0_hbm, z_hbm, s,
                   gidx2, sidx2, rows_a, rows_b, acc, gs_a, gs_b, ss_a, ss_b)
        _spmv_pass(src_hbm, dst_hbm, t1_hbm, o1_hbm, z_hbm, s,
                   gidx2, sidx2, rows_a, rows_b, acc, gs_a, gs_b, ss_a, ss_b)

    @pl.when(c == 1)
    def _():
        _spmv_pass(src_hbm, dst_hbm, t2_hbm, o2_hbm, z_hbm, s,
                   gidx2, sidx2, rows_a, rows_b, acc, gs_a, gs_b, ss_a, ss_b)
        _spmv_pass(src_hbm, dst_hbm, t3_hbm, o3_hbm, z_hbm, s,
                   gidx2, sidx2, rows_a, rows_b, acc, gs_a, gs_b, ss_a, ss_b)


def _k2_body(cnt0_ref, cnt1_ref, x_ref, dinv_ref, xs0_ref, xs1_ref):
    i = pl.program_id(0)
    cnt = cnt0_ref[:, 0:1] + cnt1_ref[:, 0:1]
    deg = cnt + 1.0
    y = lax.rsqrt(deg)
    dinv = y * (1.5 - 0.5 * deg * y * y)
    rows = i * R + lax.broadcasted_iota(jnp.int32, (R, 1), 0)
    dinv = jnp.where(rows < N, dinv, 0.0)
    dinv_ref[...] = jnp.broadcast_to(dinv, (R, CHUNK))
    xs = x_ref[...] * dinv
    xs0_ref[...] = xs[:, :CHUNK]
    xs1_ref[...] = xs[:, CHUNK:]


def _k2_call(cnt0, cnt1, xp):
    return pl.pallas_call(
        _k2_body,
        grid=(NPAD // R,),
        in_specs=[
            pl.BlockSpec((R, CHUNK), lambda i: (i, 0)),
            pl.BlockSpec((R, CHUNK), lambda i: (i, 0)),
            pl.BlockSpec((R, IN_DIM), lambda i: (i, 0)),
        ],
        out_specs=[
            pl.BlockSpec((R, CHUNK), lambda i: (i, 0)),
            pl.BlockSpec((R, CHUNK), lambda i: (i, 0)),
            pl.BlockSpec((R, CHUNK), lambda i: (i, 0)),
        ],
        out_shape=[
            jax.ShapeDtypeStruct((NPAD, CHUNK), F32),
            jax.ShapeDtypeStruct((NPAD, CHUNK), F32),
            jax.ShapeDtypeStruct((NPAD, CHUNK), F32),
        ],
    )(cnt0, cnt1, xp)


def _k4_body(p0_ref, p1_ref, xs0_ref, xs1_ref, dinv_ref, w1_ref, b1_ref,
             q0_ref, q1_ref, q2_ref, q3_ref):
    dinv = dinv_ref[:, 0:1]
    a = jnp.concatenate(
        [p0_ref[...] + xs0_ref[...], p1_ref[...] + xs1_ref[...]], axis=1)
    a = a * dinv
    h = lax.dot_general(a, w1_ref[...], (((1,), (0,)), ((), ())),
                        preferred_element_type=F32)
    h = jnp.maximum(h + b1_ref[...], 0.0)
    hs = h * dinv
    q0_ref[...] = hs[:, 0:128]
    q1_ref[...] = hs[:, 128:256]
    q2_ref[...] = hs[:, 256:384]
    q3_ref[...] = hs[:, 384:512]


def _k4_call(p0, p1, xs0, xs1, dinv, W1, b1r):
    return pl.pallas_call(
        _k4_body,
        grid=(NPAD // R,),
        in_specs=[
            pl.BlockSpec((R, CHUNK), lambda i: (i, 0)),
            pl.BlockSpec((R, CHUNK), lambda i: (i, 0)),
            pl.BlockSpec((R, CHUNK), lambda i: (i, 0)),
            pl.BlockSpec((R, CHUNK), lambda i: (i, 0)),
            pl.BlockSpec((R, CHUNK), lambda i: (i, 0)),
            pl.BlockSpec((IN_DIM, HID), lambda i: (0, 0)),
            pl.BlockSpec((1, HID), lambda i: (0, 0)),
        ],
        out_specs=[pl.BlockSpec((R, CHUNK), lambda i: (i, 0))] * 4,
        out_shape=[jax.ShapeDtypeStruct((NPAD, CHUNK), F32)] * 4,
    )(p0, p1, xs0, xs1, dinv, W1, b1r)


def _k6_body(m0_ref, m1_ref, m2_ref, m3_ref, q0_ref, q1_ref, q2_ref, q3_ref,
             dinv_ref, cp0_ref, cp1_ref, w2_ref, b2_ref, w3_ref, b3_ref,
             u_ref, racc):
    i = pl.program_id(0)
    dinv = dinv_ref[:, 0:1]
    a = jnp.concatenate([
        m0_ref[...] + q0_ref[...], m1_ref[...] + q1_ref[...],
        m2_ref[...] + q2_ref[...], m3_ref[...] + q3_ref[...]], axis=1)
    a = a * dinv
    h = lax.dot_general(a, w2_ref[...], (((1,), (0,)), ((), ())),
                        preferred_element_type=F32)
    h = jnp.maximum(h + b2_ref[...], 0.0)
    cvec = dinv * (cp0_ref[:, 0:1] + cp1_ref[:, 0:1] + dinv)
    part = lax.dot_general(cvec, h, (((0,), (0,)), ((), ())),
                           precision=HIGHEST, preferred_element_type=F32)

    @pl.when(i == 0)
    def _():
        racc[...] = part

    @pl.when(i > 0)
    def _():
        racc[...] = racc[...] + part

    @pl.when(i == pl.num_programs(0) - 1)
    def _():
        r = racc[...] * (1.0 / N)
        w3q = w3_ref[...].astype(jnp.bfloat16).astype(F32)
        u_ref[...] = lax.dot_general(r, w3q, (((1,), (0,)), ((), ())),
                                     precision=HIGHEST,
                                     preferred_element_type=F32) + b3_ref[...]


def _k6_call(m0, m1, m2, m3, q0, q1, q2, q3, dinv, cp0, cp1, W2, b2r, W3, b3r):
    return pl.pallas_call(
        _k6_body,
        grid=(NPAD // R,),
        in_specs=[pl.BlockSpec((R, CHUNK), lambda i: (i, 0))] * 8 + [
            pl.BlockSpec((R, CHUNK), lambda i: (i, 0)),
            pl.BlockSpec((R, CHUNK), lambda i: (i, 0)),
            pl.BlockSpec((R, CHUNK), lambda i: (i, 0)),
            pl.BlockSpec((HID, HID), lambda i: (0, 0)),
            pl.BlockSpec((1, HID), lambda i: (0, 0)),
            pl.BlockSpec((HID, HID), lambda i: (0, 0)),
            pl.BlockSpec((1, HID), lambda i: (0, 0)),
        ],
        out_specs=pl.BlockSpec((1, HID), lambda i: (0, 0)),
        out_shape=jax.ShapeDtypeStruct((1, HID), F32),
        scratch_shapes=[pltpu.VMEM((1, HID), F32)],
    )(m0, m1, m2, m3, q0, q1, q2, q3, dinv, cp0, cp1, W2, b2r, W3, b3r)


def kernel(x, edge_index, W1, b1, W2, b2, W3, b3):
    src = edge_index[0].astype(jnp.int32)
    dst = edge_index[1].astype(jnp.int32)
    padfill = jnp.full((EPAD - E,), N, jnp.int32)
    srcp = jnp.concatenate([src, padfill]).reshape(EPAD // CHUNK, CHUNK)
    dstp = jnp.concatenate([dst, padfill]).reshape(EPAD // CHUNK, CHUNK)
    xp = jnp.zeros((NPAD, IN_DIM), F32).at[:N].set(x)
    zeros128 = jnp.zeros((CHUNK, CHUNK), F32)
    ones128 = jnp.ones((CHUNK, CHUNK), F32)
    b1r = b1.reshape(1, HID)
    b2r = b2.reshape(1, HID)
    b3r = b3.reshape(1, HID)

    cnt0, cnt1 = _k1_degree()(dstp, ones128, zeros128)
    dinv, xs0, xs1 = _k2_call(cnt0, cnt1, xp)
    p0, p1 = _k3_spmv2()(srcp, dstp, xs0, xs1, zeros128)
    cp0, cp1 = _kc_cvec()(srcp, dstp, dinv, zeros128)
    q0, q1, q2, q3 = _k4_call(p0, p1, xs0, xs1, dinv, W1, b1r)
    m0, m1, m2, m3 = _k5_spmv4()(srcp, dstp, q0, q1, q2, q3, zeros128)
    u = _k6_call(m0, m1, m2, m3, q0, q1, q2, q3, dinv, cp0, cp1,
                 W2, b2r, W3, b3r)
    return u
